# baseline (speedup 1.0000x reference)
import dataclasses
import functools

import jax
import jax.numpy as jnp
from jax import lax
from jax.experimental import pallas as pl
from jax.experimental.pallas import tpu as pltpu
from jax.experimental.pallas import tpu_sc as plsc

N = 10000
E = 320000
IN_CH = 128
H = 32
NC = 2
NS = 16
NW = NC * NS
EPW = E // NW
EPC = E // NS
CH = 125
NCHW = EPW // CH
NPAD = 10240
RPT = NPAD // NS
OPT = NPAD // NW

_mesh = plsc.VectorSubcoreMesh(core_axis_name="c", subcore_axis_name="s")

_sc_params = pltpu.CompilerParams()
if "needs_layout_passes" in pltpu.CompilerParams.__dataclass_fields__:
    _sc_params = dataclasses.replace(_sc_params, needs_layout_passes=False)
_sc_params_untiled = dataclasses.replace(_sc_params, use_tc_tiling_on_sc=False)


def _rsqrt16(d):
    bits = plsc.bitcast(d, jnp.int32)
    y = plsc.bitcast(jnp.int32(0x5F3759DF) - (bits >> 1), jnp.float32)
    for _ in range(3):
        y = y * (1.5 - 0.5 * d * y * y)
    return y


def _sc_deg_dinv(dst):
    @functools.partial(
        pl.kernel,
        out_type=jax.ShapeDtypeStruct((NPAD,), jnp.float32),
        mesh=_mesh,
        compiler_params=_sc_params,
        scratch_types=[
            pltpu.VMEM((EPC,), jnp.int32),
            pltpu.VMEM((NPAD,), jnp.float32),
            pltpu.VMEM((RPT,), jnp.float32),
            pltpu.VMEM((RPT,), jnp.float32),
            pltpu.VMEM_SHARED((NS, NPAD), jnp.float32),
        ],
    )
    def k(dst_hbm, out_hbm, dst_v, hist_v, buf_v, acc_v, shared):
        c = lax.axis_index("c")
        s = lax.axis_index("s")

        @pl.when(c == 0)
        def _():
            pltpu.sync_copy(dst_hbm.at[pl.ds(s * EPC, EPC)], dst_v)

            @pl.loop(0, NPAD, step=16)
            def _(i):
                hist_v[pl.ds(i, 16)] = jnp.zeros((16,), jnp.float32)

            ones16 = jnp.full((16,), 1.0, jnp.float32)

            @pl.loop(0, EPC, step=32)
            def _(i):
                dv0 = dst_v[pl.ds(i, 16)]
                dv1 = dst_v[pl.ds(i + 16, 16)]
                plsc.addupdate_scatter(hist_v, [dv0], ones16)
                plsc.addupdate_scatter(hist_v, [dv1], ones16)

            pltpu.sync_copy(hist_v, shared.at[s])
            plsc.subcore_barrier()

            pltpu.sync_copy(shared.at[0, pl.ds(s * RPT, RPT)], acc_v)

            @pl.loop(1, NS)
            def _(kk):
                pltpu.sync_copy(shared.at[kk, pl.ds(s * RPT, RPT)], buf_v)

                @pl.loop(0, RPT, step=16)
                def _(r):
                    acc_v[pl.ds(r, 16)] = acc_v[pl.ds(r, 16)] + buf_v[pl.ds(r, 16)]

            @pl.loop(0, RPT, step=16)
            def _(r):
                acc_v[pl.ds(r, 16)] = _rsqrt16(acc_v[pl.ds(r, 16)] + 1.0)

            pltpu.sync_copy(acc_v, out_hbm.at[pl.ds(s * RPT, RPT)])

    return k(dst)


def _sc_row_scatter(p, src_r, dst_r, zeros):
    NBUF = 8
    AHEAD = 4

    @functools.partial(
        pl.kernel,
        out_type=jax.ShapeDtypeStruct((NC, NPAD, H), jnp.float32),
        mesh=_mesh,
        compiler_params=_sc_params_untiled,
        scratch_types=[
            pltpu.VMEM((NCHW, CH), jnp.int32),
            pltpu.VMEM((NCHW, CH), jnp.int32),
            pltpu.VMEM((RPT, H), jnp.float32),
            pltpu.VMEM_SHARED((NPAD, H), jnp.float32),
        ]
        + [pltpu.VMEM((CH, H), jnp.float32) for _ in range(NBUF)]
        + [pltpu.SemaphoreType.DMA for _ in range(2 * NBUF)],
    )
    def k(p_hbm, srcr_hbm, dstr_hbm, z_hbm, out_hbm,
          idx_s, idx_d, z_v, acc_sh, *rest):
        bufs = rest[:NBUF]
        gs = rest[NBUF:2 * NBUF]
        ss = rest[2 * NBUF:]
        c = lax.axis_index("c")
        s = lax.axis_index("s")
        w = c * NS + s
        pltpu.sync_copy(srcr_hbm.at[pl.ds(w * NCHW, NCHW)], idx_s)
        pltpu.sync_copy(dstr_hbm.at[pl.ds(w * NCHW, NCHW)], idx_d)

        pltpu.sync_copy(z_hbm.at[pl.ds(s * RPT, RPT)], z_v)
        pltpu.sync_copy(z_v, acc_sh.at[pl.ds(s * RPT, RPT)])
        plsc.subcore_barrier()

        for kk in range(AHEAD):
            pltpu.async_copy(p_hbm.at[idx_s.at[kk]], bufs[kk], gs[kk])

        @pl.loop(0, NCHW, step=NBUF)
        def _(j):
            for kk in range(NBUF):
                m = j + kk
                pltpu.make_async_copy(p_hbm.at[idx_s.at[m]], bufs[kk],
                                      gs[kk]).wait()
                pltpu.async_copy(bufs[kk], acc_sh.at[idx_d.at[m]], ss[kk],
                                 add=True)
                fb = (kk + AHEAD) % NBUF

                @pl.when(m + AHEAD < NCHW)
                def _():
                    @pl.when(m >= NBUF - AHEAD)
                    def _():
                        pltpu.make_async_copy(
                            bufs[fb], acc_sh.at[idx_d.at[m - (NBUF - AHEAD)]],
                            ss[fb]).wait()

                    pltpu.async_copy(p_hbm.at[idx_s.at[m + AHEAD]], bufs[fb],
                                     gs[fb])

        for m in range(NCHW - NBUF, NCHW):
            fb = m % NBUF
            pltpu.make_async_copy(bufs[fb], acc_sh.at[idx_d.at[m]],
                                  ss[fb]).wait()

        plsc.subcore_barrier()
        pltpu.sync_copy(acc_sh.at[pl.ds(s * RPT, RPT)], z_v)
        pltpu.sync_copy(z_v, out_hbm.at[c, pl.ds(s * RPT, RPT)])

    return k(p, src_r, dst_r, zeros)


def _sc_scalar_final(Sp, p, dinv, src, dst, b1r, w2r, b2r):
    HOPT = RPT // 2

    @functools.partial(
        pl.kernel,
        out_type=jax.ShapeDtypeStruct((NPAD,), jnp.float32),
        mesh=_mesh,
        compiler_params=_sc_params_untiled,
        scratch_types=[
            pltpu.VMEM((NPAD,), jnp.float32),
            pltpu.VMEM((EPC,), jnp.int32),
            pltpu.VMEM((EPC,), jnp.int32),
            pltpu.VMEM((NPAD,), jnp.float32),
            pltpu.VMEM((RPT,), jnp.float32),
            pltpu.VMEM((RPT,), jnp.float32),
            pltpu.VMEM((OPT,), jnp.float32),
            pltpu.VMEM((OPT,), jnp.float32),
            pltpu.VMEM((RPT,), jnp.float32),
            pltpu.VMEM((RPT,), jnp.float32),
            pltpu.VMEM((HOPT * H,), jnp.float32),
            pltpu.VMEM((HOPT * H,), jnp.float32),
            pltpu.VMEM((HOPT * H,), jnp.float32),
            pltpu.VMEM((H * 16,), jnp.float32),
            pltpu.VMEM((H * 16,), jnp.float32),
            pltpu.VMEM((16,), jnp.float32),
            pltpu.VMEM_SHARED((NS, NPAD), jnp.float32),
            pltpu.VMEM_SHARED((NPAD,), jnp.float32),
        ],
    )
    def k(sp_hbm, p_hbm, dinv_hbm, src_hbm, dst_hbm, b1_hbm, w2_hbm, b2_hbm,
          out_hbm,
          q_v, src_v, dst_v, hist_v, buf_v, acc_v, dv_v, t_v,
          dq_v, ql_v, s0_v, s1_v, p_v, b1s_v, w2s_v, b2_v, shared, qsh):
        c = lax.axis_index("c")
        s = lax.axis_index("s")
        base = c * (NPAD // NC) + s * OPT
        qbase = s * RPT
        pltpu.sync_copy(src_hbm.at[pl.ds(s * EPC, EPC)], src_v)
        pltpu.sync_copy(dst_hbm.at[pl.ds(s * EPC, EPC)], dst_v)
        pltpu.sync_copy(b1_hbm, b1s_v)
        pltpu.sync_copy(w2_hbm, w2s_v)
        pltpu.sync_copy(b2_hbm, b2_v)
        pltpu.sync_copy(dinv_hbm.at[pl.ds(base, OPT)], dv_v)
        pltpu.sync_copy(dinv_hbm.at[pl.ds(qbase, RPT)], dq_v)

        lane_stride = lax.iota(jnp.int32, 16) * H

        for half in range(2):
            off = qbase + half * HOPT
            pltpu.sync_copy(sp_hbm.at[0, pl.ds(off * H, HOPT * H)], s0_v)
            pltpu.sync_copy(sp_hbm.at[1, pl.ds(off * H, HOPT * H)], s1_v)
            pltpu.sync_copy(p_hbm.at[pl.ds(off * H, HOPT * H)], p_v)

            @pl.loop(0, HOPT, step=16)
            def _(g, _half=half):
                gH = g * H
                dinv16 = dq_v[pl.ds(_half * HOPT + g, 16)]
                acc = jnp.zeros((16,), jnp.float32)
                for col in range(H):
                    cv = lane_stride + (gH + col)
                    sv = (plsc.load_gather(s0_v, [cv])
                          + plsc.load_gather(s1_v, [cv])
                          + plsc.load_gather(p_v, [cv]))
                    h1 = jnp.maximum(dinv16 * sv + b1s_v[pl.ds(col * 16, 16)],
                                     0.0)
                    acc = acc + h1 * w2s_v[pl.ds(col * 16, 16)]
                ql_v[pl.ds(_half * HOPT + g, 16)] = acc * dinv16

        pltpu.sync_copy(ql_v, qsh.at[pl.ds(qbase, RPT)])
        plsc.subcore_barrier()
        pltpu.sync_copy(qsh, q_v)

        @pl.loop(0, NPAD, step=16)
        def _(i):
            hist_v[pl.ds(i, 16)] = jnp.zeros((16,), jnp.float32)

        @pl.loop(0, EPC, step=32)
        def _(i):
            sv0 = src_v[pl.ds(i, 16)]
            dv0 = dst_v[pl.ds(i, 16)]
            sv1 = src_v[pl.ds(i + 16, 16)]
            dv1 = dst_v[pl.ds(i + 16, 16)]
            vals0 = plsc.load_gather(q_v, [sv0])
            plsc.addupdate_scatter(hist_v, [dv0], vals0)
            vals1 = plsc.load_gather(q_v, [sv1])
            plsc.addupdate_scatter(hist_v, [dv1], vals1)

        pltpu.sync_copy(hist_v, shared.at[s])
        plsc.subcore_barrier()

        pltpu.sync_copy(shared.at[0, pl.ds(s * RPT, RPT)], acc_v)

        @pl.loop(1, NS)
        def _(kk):
            pltpu.sync_copy(shared.at[kk, pl.ds(s * RPT, RPT)], buf_v)

            @pl.loop(0, RPT, step=16)
            def _(r):
                acc_v[pl.ds(r, 16)] = acc_v[pl.ds(r, 16)] + buf_v[pl.ds(r, 16)]

        pltpu.sync_copy(acc_v, qsh.at[pl.ds(s * RPT, RPT)])
        plsc.subcore_barrier()
        pltpu.sync_copy(qsh.at[pl.ds(base, OPT)], t_v)

        b2v = b2_v[pl.ds(0, 16)]

        @pl.loop(0, OPT, step=16)
        def _(r):
            qv = q_v[pl.ds(base + r, 16)]
            z = dv_v[pl.ds(r, 16)] * (t_v[pl.ds(r, 16)] + qv) + b2v
            den = 1.0 + jnp.exp(-z)
            rec = 1.0 / den
            rec = rec * (2.0 - den * rec)
            rec = rec * (2.0 - den * rec)
            t_v[pl.ds(r, 16)] = rec

        pltpu.sync_copy(t_v, out_hbm.at[pl.ds(base, OPT)])

    return k(Sp, p, dinv, src, dst, b1r, w2r, b2r)


def _tc_prep(x_pad, W1, dinv):
    R = 1024

    def body(x_ref, w_ref, dinv_ref, p_ref):
        h = jnp.dot(x_ref[...], w_ref[...], preferred_element_type=jnp.float32)
        p_ref[...] = h * dinv_ref[...]

    return pl.pallas_call(
        body,
        grid=(NPAD // R,),
        in_specs=[
            pl.BlockSpec((R, IN_CH), lambda i: (i, 0)),
            pl.BlockSpec((IN_CH, H), lambda i: (0, 0)),
            pl.BlockSpec((R, 1), lambda i: (i, 0)),
        ],
        out_specs=pl.BlockSpec((R, H), lambda i: (i, 0)),
        out_shape=jax.ShapeDtypeStruct((NPAD, H), jnp.float32),
    )(x_pad, W1, dinv)


def kernel(x, edge_index, W1, b1, W2, b2):
    src = edge_index[0]
    dst = edge_index[1]
    src_r = src.reshape(NW * NCHW, CH)
    dst_r = dst.reshape(NW * NCHW, CH)

    dinv_pad = _sc_deg_dinv(dst)
    x_pad = jnp.pad(x, ((0, NPAD - N), (0, 0)))
    p = _tc_prep(x_pad, W1, dinv_pad.reshape(NPAD, 1))

    zeros = jnp.zeros((NPAD, H), jnp.float32)
    Sp = _sc_row_scatter(p, src_r, dst_r, zeros)

    b2r = jnp.broadcast_to(b2.reshape(1), (16,))
    b1s = jnp.broadcast_to(b1.reshape(H, 1), (H, 16)).reshape(H * 16)
    w2s = jnp.broadcast_to(W2.reshape(H, 1), (H, 16)).reshape(H * 16)
    out_pad = _sc_scalar_final(Sp.reshape(NC, NPAD * H), p.reshape(NPAD * H),
                               dinv_pad, src, dst, b1s, w2s, b2r)
    return out_pad[:N].reshape(N, 1)


# speedup vs baseline: 1.1352x; 1.1352x over previous
import dataclasses
import functools

import jax
import jax.numpy as jnp
from jax import lax
from jax.experimental import pallas as pl
from jax.experimental.pallas import tpu as pltpu
from jax.experimental.pallas import tpu_sc as plsc

N = 10000
E = 320000
IN_CH = 128
H = 32
NC = 2
NS = 16
NW = NC * NS
EPW = E // NW
EPC = E // NS
CH = 125
NCHW = EPW // CH
NPAD = 10240
RPT = NPAD // NS
OPT = NPAD // NW

_mesh = plsc.VectorSubcoreMesh(core_axis_name="c", subcore_axis_name="s")

_sc_params = pltpu.CompilerParams()
if "needs_layout_passes" in pltpu.CompilerParams.__dataclass_fields__:
    _sc_params = dataclasses.replace(_sc_params, needs_layout_passes=False)
_sc_params_untiled = dataclasses.replace(_sc_params, use_tc_tiling_on_sc=False)


def _rsqrt16(d):
    bits = plsc.bitcast(d, jnp.int32)
    y = plsc.bitcast(jnp.int32(0x5F3759DF) - (bits >> 1), jnp.float32)
    for _ in range(3):
        y = y * (1.5 - 0.5 * d * y * y)
    return y


def _sc_deg_dinv(dst):
    @functools.partial(
        pl.kernel,
        out_type=jax.ShapeDtypeStruct((NPAD,), jnp.float32),
        mesh=_mesh,
        compiler_params=_sc_params,
        scratch_types=[
            pltpu.VMEM((EPC,), jnp.int32),
            pltpu.VMEM((NPAD,), jnp.float32),
            pltpu.VMEM((RPT,), jnp.float32),
            pltpu.VMEM((RPT,), jnp.float32),
            pltpu.VMEM_SHARED((NS, NPAD), jnp.float32),
        ],
    )
    def k(dst_hbm, out_hbm, dst_v, hist_v, buf_v, acc_v, shared):
        c = lax.axis_index("c")
        s = lax.axis_index("s")

        @pl.when(c == 0)
        def _():
            pltpu.sync_copy(dst_hbm.at[pl.ds(s * EPC, EPC)], dst_v)

            @pl.loop(0, NPAD, step=16)
            def _(i):
                hist_v[pl.ds(i, 16)] = jnp.zeros((16,), jnp.float32)

            ones16 = jnp.full((16,), 1.0, jnp.float32)

            @pl.loop(0, EPC, step=32)
            def _(i):
                dv0 = dst_v[pl.ds(i, 16)]
                dv1 = dst_v[pl.ds(i + 16, 16)]
                plsc.addupdate_scatter(hist_v, [dv0], ones16)
                plsc.addupdate_scatter(hist_v, [dv1], ones16)

            pltpu.sync_copy(hist_v, shared.at[s])
            plsc.subcore_barrier()

            pltpu.sync_copy(shared.at[0, pl.ds(s * RPT, RPT)], acc_v)

            @pl.loop(1, NS)
            def _(kk):
                pltpu.sync_copy(shared.at[kk, pl.ds(s * RPT, RPT)], buf_v)

                @pl.loop(0, RPT, step=16)
                def _(r):
                    acc_v[pl.ds(r, 16)] = acc_v[pl.ds(r, 16)] + buf_v[pl.ds(r, 16)]

            @pl.loop(0, RPT, step=16)
            def _(r):
                acc_v[pl.ds(r, 16)] = _rsqrt16(acc_v[pl.ds(r, 16)] + 1.0)

            pltpu.sync_copy(acc_v, out_hbm.at[pl.ds(s * RPT, RPT)])

    return k(dst)


def _sc_row_scatter(p, src_r, dst_r, zeros):
    NBUF = 8
    AHEAD = 4

    @functools.partial(
        pl.kernel,
        out_type=jax.ShapeDtypeStruct((NC, NPAD, H), jnp.float32),
        mesh=_mesh,
        compiler_params=_sc_params_untiled,
        scratch_types=[
            pltpu.VMEM((NCHW, CH), jnp.int32),
            pltpu.VMEM((NCHW, CH), jnp.int32),
            pltpu.VMEM((RPT, H), jnp.float32),
            pltpu.VMEM_SHARED((NPAD, H), jnp.float32),
        ]
        + [pltpu.VMEM((CH, H), jnp.float32) for _ in range(NBUF)]
        + [pltpu.SemaphoreType.DMA for _ in range(2 * NBUF)],
    )
    def k(p_hbm, srcr_hbm, dstr_hbm, z_hbm, out_hbm,
          idx_s, idx_d, z_v, acc_sh, *rest):
        bufs = rest[:NBUF]
        gs = rest[NBUF:2 * NBUF]
        ss = rest[2 * NBUF:]
        c = lax.axis_index("c")
        s = lax.axis_index("s")
        w = c * NS + s
        pltpu.sync_copy(srcr_hbm.at[pl.ds(w * NCHW, NCHW)], idx_s)
        pltpu.sync_copy(dstr_hbm.at[pl.ds(w * NCHW, NCHW)], idx_d)

        pltpu.sync_copy(z_hbm.at[pl.ds(s * RPT, RPT)], z_v)
        pltpu.sync_copy(z_v, acc_sh.at[pl.ds(s * RPT, RPT)])
        plsc.subcore_barrier()

        for kk in range(AHEAD):
            pltpu.async_copy(p_hbm.at[idx_s.at[kk]], bufs[kk], gs[kk])

        @pl.loop(0, NCHW, step=NBUF)
        def _(j):
            for kk in range(NBUF):
                m = j + kk
                pltpu.make_async_copy(p_hbm.at[idx_s.at[m]], bufs[kk],
                                      gs[kk]).wait()
                pltpu.async_copy(bufs[kk], acc_sh.at[idx_d.at[m]], ss[kk],
                                 add=True)
                fb = (kk + AHEAD) % NBUF

                @pl.when(m + AHEAD < NCHW)
                def _():
                    @pl.when(m >= NBUF - AHEAD)
                    def _():
                        pltpu.make_async_copy(
                            bufs[fb], acc_sh.at[idx_d.at[m - (NBUF - AHEAD)]],
                            ss[fb]).wait()

                    pltpu.async_copy(p_hbm.at[idx_s.at[m + AHEAD]], bufs[fb],
                                     gs[fb])

        for m in range(NCHW - NBUF, NCHW):
            fb = m % NBUF
            pltpu.make_async_copy(bufs[fb], acc_sh.at[idx_d.at[m]],
                                  ss[fb]).wait()

        plsc.subcore_barrier()
        pltpu.sync_copy(acc_sh.at[pl.ds(s * RPT, RPT)], z_v)
        pltpu.sync_copy(z_v, out_hbm.at[c, pl.ds(s * RPT, RPT)])

    return k(p, src_r, dst_r, zeros)


def _sc_scalar_final(Sp, p, dinv, src, dst, b1r, w2r, b2r):
    HOPT = RPT // 2

    @functools.partial(
        pl.kernel,
        out_type=jax.ShapeDtypeStruct((NPAD,), jnp.float32),
        mesh=_mesh,
        compiler_params=_sc_params_untiled,
        scratch_types=[
            pltpu.VMEM((NPAD,), jnp.float32),
            pltpu.VMEM((EPC,), jnp.int32),
            pltpu.VMEM((EPC,), jnp.int32),
            pltpu.VMEM((NPAD,), jnp.float32),
            pltpu.VMEM((RPT,), jnp.float32),
            pltpu.VMEM((RPT,), jnp.float32),
            pltpu.VMEM((OPT,), jnp.float32),
            pltpu.VMEM((OPT,), jnp.float32),
            pltpu.VMEM((RPT,), jnp.float32),
            pltpu.VMEM((RPT,), jnp.float32),
            pltpu.VMEM((HOPT * H,), jnp.float32),
            pltpu.VMEM((HOPT * H,), jnp.float32),
            pltpu.VMEM((HOPT * H,), jnp.float32),
            pltpu.VMEM((H * 16,), jnp.float32),
            pltpu.VMEM((H * 16,), jnp.float32),
            pltpu.VMEM((16,), jnp.float32),
            pltpu.VMEM_SHARED((NS, NPAD), jnp.float32),
            pltpu.VMEM_SHARED((NPAD,), jnp.float32),
        ],
    )
    def k(sp_hbm, p_hbm, dinv_hbm, src_hbm, dst_hbm, b1_hbm, w2_hbm, b2_hbm,
          out_hbm,
          q_v, src_v, dst_v, hist_v, buf_v, acc_v, dv_v, t_v,
          dq_v, ql_v, s0_v, s1_v, p_v, b1s_v, w2s_v, b2_v, shared, qsh):
        c = lax.axis_index("c")
        s = lax.axis_index("s")
        base = c * (NPAD // NC) + s * OPT
        qbase = s * RPT
        pltpu.sync_copy(src_hbm.at[pl.ds(s * EPC, EPC)], src_v)
        pltpu.sync_copy(dst_hbm.at[pl.ds(s * EPC, EPC)], dst_v)
        pltpu.sync_copy(b1_hbm, b1s_v)
        pltpu.sync_copy(w2_hbm, w2s_v)
        pltpu.sync_copy(b2_hbm, b2_v)
        pltpu.sync_copy(dinv_hbm.at[pl.ds(base, OPT)], dv_v)
        pltpu.sync_copy(dinv_hbm.at[pl.ds(qbase, RPT)], dq_v)

        lane_stride = lax.iota(jnp.int32, 16) * H

        for half in range(2):
            off = qbase + half * HOPT
            pltpu.sync_copy(sp_hbm.at[0, pl.ds(off * H, HOPT * H)], s0_v)
            pltpu.sync_copy(sp_hbm.at[1, pl.ds(off * H, HOPT * H)], s1_v)
            pltpu.sync_copy(p_hbm.at[pl.ds(off * H, HOPT * H)], p_v)

            @pl.loop(0, HOPT * H, step=64)
            def _(i):
                for u in range(0, 64, 16):
                    s0_v[pl.ds(i + u, 16)] = (s0_v[pl.ds(i + u, 16)]
                                              + s1_v[pl.ds(i + u, 16)]
                                              + p_v[pl.ds(i + u, 16)])

            @pl.loop(0, HOPT, step=16)
            def _(g, _half=half):
                gH = g * H
                dinv16 = dq_v[pl.ds(_half * HOPT + g, 16)]
                acc = jnp.zeros((16,), jnp.float32)
                acc2 = jnp.zeros((16,), jnp.float32)
                for col in range(0, H, 2):
                    cv = lane_stride + (gH + col)
                    cv2 = lane_stride + (gH + col + 1)
                    sv = plsc.load_gather(s0_v, [cv])
                    sv2 = plsc.load_gather(s0_v, [cv2])
                    h1 = jnp.maximum(dinv16 * sv + b1s_v[pl.ds(col * 16, 16)],
                                     0.0)
                    h2 = jnp.maximum(
                        dinv16 * sv2 + b1s_v[pl.ds((col + 1) * 16, 16)], 0.0)
                    acc = acc + h1 * w2s_v[pl.ds(col * 16, 16)]
                    acc2 = acc2 + h2 * w2s_v[pl.ds((col + 1) * 16, 16)]
                ql_v[pl.ds(_half * HOPT + g, 16)] = (acc + acc2) * dinv16

        pltpu.sync_copy(ql_v, qsh.at[pl.ds(qbase, RPT)])
        plsc.subcore_barrier()
        pltpu.sync_copy(qsh, q_v)

        @pl.loop(0, NPAD, step=16)
        def _(i):
            hist_v[pl.ds(i, 16)] = jnp.zeros((16,), jnp.float32)

        @pl.loop(0, EPC, step=32)
        def _(i):
            sv0 = src_v[pl.ds(i, 16)]
            dv0 = dst_v[pl.ds(i, 16)]
            sv1 = src_v[pl.ds(i + 16, 16)]
            dv1 = dst_v[pl.ds(i + 16, 16)]
            vals0 = plsc.load_gather(q_v, [sv0])
            plsc.addupdate_scatter(hist_v, [dv0], vals0)
            vals1 = plsc.load_gather(q_v, [sv1])
            plsc.addupdate_scatter(hist_v, [dv1], vals1)

        pltpu.sync_copy(hist_v, shared.at[s])
        plsc.subcore_barrier()

        pltpu.sync_copy(shared.at[0, pl.ds(s * RPT, RPT)], acc_v)

        @pl.loop(1, NS)
        def _(kk):
            pltpu.sync_copy(shared.at[kk, pl.ds(s * RPT, RPT)], buf_v)

            @pl.loop(0, RPT, step=16)
            def _(r):
                acc_v[pl.ds(r, 16)] = acc_v[pl.ds(r, 16)] + buf_v[pl.ds(r, 16)]

        pltpu.sync_copy(acc_v, qsh.at[pl.ds(s * RPT, RPT)])
        plsc.subcore_barrier()
        pltpu.sync_copy(qsh.at[pl.ds(base, OPT)], t_v)

        b2v = b2_v[pl.ds(0, 16)]

        @pl.loop(0, OPT, step=16)
        def _(r):
            qv = q_v[pl.ds(base + r, 16)]
            z = dv_v[pl.ds(r, 16)] * (t_v[pl.ds(r, 16)] + qv) + b2v
            den = 1.0 + jnp.exp(-z)
            rec = 1.0 / den
            rec = rec * (2.0 - den * rec)
            rec = rec * (2.0 - den * rec)
            t_v[pl.ds(r, 16)] = rec

        pltpu.sync_copy(t_v, out_hbm.at[pl.ds(base, OPT)])

    return k(Sp, p, dinv, src, dst, b1r, w2r, b2r)


def _tc_prep(x_pad, W1, dinv):
    R = 1024

    def body(x_ref, w_ref, dinv_ref, p_ref):
        h = jnp.dot(x_ref[...], w_ref[...], preferred_element_type=jnp.float32)
        p_ref[...] = h * dinv_ref[...]

    return pl.pallas_call(
        body,
        grid=(NPAD // R,),
        in_specs=[
            pl.BlockSpec((R, IN_CH), lambda i: (i, 0)),
            pl.BlockSpec((IN_CH, H), lambda i: (0, 0)),
            pl.BlockSpec((R, 1), lambda i: (i, 0)),
        ],
        out_specs=pl.BlockSpec((R, H), lambda i: (i, 0)),
        out_shape=jax.ShapeDtypeStruct((NPAD, H), jnp.float32),
    )(x_pad, W1, dinv)


def kernel(x, edge_index, W1, b1, W2, b2):
    src = edge_index[0]
    dst = edge_index[1]
    src_r = src.reshape(NW * NCHW, CH)
    dst_r = dst.reshape(NW * NCHW, CH)

    dinv_pad = _sc_deg_dinv(dst)
    x_pad = jnp.pad(x, ((0, NPAD - N), (0, 0)))
    p = _tc_prep(x_pad, W1, dinv_pad.reshape(NPAD, 1))

    zeros = jnp.zeros((NPAD, H), jnp.float32)
    Sp = _sc_row_scatter(p, src_r, dst_r, zeros)

    b2r = jnp.broadcast_to(b2.reshape(1), (16,))
    b1s = jnp.broadcast_to(b1.reshape(H, 1), (H, 16)).reshape(H * 16)
    w2s = jnp.broadcast_to(W2.reshape(H, 1), (H, 16)).reshape(H * 16)
    out_pad = _sc_scalar_final(Sp.reshape(NC, NPAD * H), p.reshape(NPAD * H),
                               dinv_pad, src, dst, b1s, w2s, b2r)
    return out_pad[:N].reshape(N, 1)
